# baseline (speedup 1.0000x reference)
import functools

import jax
import jax.numpy as jnp
from jax import lax
from jax.experimental import pallas as pl
from jax.experimental.pallas import tpu as pltpu
from jax.experimental.pallas import tpu_sc as plsc

EMB_DIM = 64
NUM_CORES = 2
NUM_SUBCORES = 16
NUM_WORKERS = NUM_CORES * NUM_SUBCORES
HC = 5
TPAD = 128


def _make_lookup(batch: int, hist: int):
    per_worker = batch // NUM_WORKERS
    chunks = hist // HC
    rows_per_chunk = HC * per_worker
    mesh = plsc.VectorSubcoreMesh(core_axis_name="c", subcore_axis_name="s")

    @functools.partial(
        pl.kernel,
        mesh=mesh,
        out_type=jax.ShapeDtypeStruct((hist * EMB_DIM, batch), jnp.float32),
        scratch_types=[
            pltpu.VMEM((hist, per_worker), jnp.int32),
            pltpu.VMEM((rows_per_chunk, EMB_DIM), jnp.float32),
            pltpu.VMEM((rows_per_chunk, EMB_DIM), jnp.float32),
            pltpu.VMEM((HC * EMB_DIM, TPAD), jnp.float32),
            pltpu.SemaphoreType.DMA,
            pltpu.SemaphoreType.DMA,
        ],
        compiler_params=pltpu.CompilerParams(
            use_tc_tiling_on_sc=False, needs_layout_passes=False
        ),
    )
    def lookup(idx_hbm, table_hbm, out_hbm, idx_v, buf_g0, buf_g1, buf_t, sem0, sem1):
        wid = lax.axis_index("s") * NUM_CORES + lax.axis_index("c")
        pltpu.sync_copy(idx_hbm.at[wid], idx_v)
        lanes = lax.iota(jnp.int32, 16)
        row_vecs = [
            [lanes + (hh * EMB_DIM + e0 * 16) for e0 in range(EMB_DIM // 16)]
            for hh in range(HC)
        ]
        bufs = [buf_g0, buf_g1]
        sems = [sem0, sem1]

        def fire(c, bg):
            for hh in range(HC):
                pltpu.async_copy(
                    table_hbm.at[idx_v.at[c * HC + hh]],
                    bufs[bg].at[pl.ds(hh * per_worker, per_worker)],
                    sems[bg],
                )

        def drain(bg):
            pltpu.make_async_copy(
                table_hbm.at[pl.ds(0, rows_per_chunk)], bufs[bg], sems[bg]
            ).wait()

        def process(c, bg):
            buf = bufs[bg]

            def tr_step(b, carry2):
                b_vec = jnp.full((16,), b, dtype=jnp.int32)
                for hh in range(HC):
                    for e0 in range(EMB_DIM // 16):
                        vals = buf[hh * per_worker + b, pl.ds(e0 * 16, 16)]
                        plsc.store_scatter(
                            buf_t, [row_vecs[hh][e0], b_vec], vals
                        )
                return carry2

            lax.fori_loop(0, per_worker, tr_step, 0)
            pltpu.sync_copy(
                buf_t,
                out_hbm.at[
                    pl.ds(c * HC * EMB_DIM, HC * EMB_DIM),
                    pl.ds(wid * per_worker, per_worker),
                ],
            )

        fire(0, 0)

        def pair_step(i, carry):
            c0 = 2 * i
            drain(0)
            fire(c0 + 1, 1)
            process(c0, 0)
            drain(1)

            @pl.when(i < chunks // 2 - 1)
            def _():
                fire(c0 + 2, 0)

            process(c0 + 1, 1)
            return carry

        lax.fori_loop(0, chunks // 2, pair_step, 0)

    return lookup


def kernel(x, table):
    batch, hist = x.shape
    idx3d = x.reshape(NUM_WORKERS, batch // NUM_WORKERS, hist).transpose(0, 2, 1)
    out = _make_lookup(batch, hist)(idx3d, table)
    return out.reshape(hist, EMB_DIM, batch).transpose(2, 0, 1)


# speedup vs baseline: 1.1176x; 1.1176x over previous
import functools

import jax
import jax.numpy as jnp
from jax import lax
from jax.experimental import pallas as pl
from jax.experimental.pallas import tpu as pltpu
from jax.experimental.pallas import tpu_sc as plsc

EMB_DIM = 64
NUM_CORES = 2
NUM_SUBCORES = 16
NUM_WORKERS = NUM_CORES * NUM_SUBCORES
HC = 5
TPAD = 137


def _make_lookup(batch: int, hist: int):
    per_worker = batch // NUM_WORKERS
    chunks = hist // HC
    rows_per_chunk = HC * per_worker
    mesh = plsc.VectorSubcoreMesh(core_axis_name="c", subcore_axis_name="s")

    @functools.partial(
        pl.kernel,
        mesh=mesh,
        out_type=jax.ShapeDtypeStruct((hist * EMB_DIM, batch), jnp.float32),
        scratch_types=[
            pltpu.VMEM((hist, per_worker), jnp.int32),
            pltpu.VMEM((rows_per_chunk, EMB_DIM), jnp.float32),
            pltpu.VMEM((HC * EMB_DIM, TPAD), jnp.float32),
            pltpu.SemaphoreType.DMA,
        ],
        compiler_params=pltpu.CompilerParams(
            use_tc_tiling_on_sc=False, needs_layout_passes=False
        ),
    )
    def lookup(idx_hbm, table_hbm, out_hbm, idx_v, buf_g, buf_t, sem):
        wid = lax.axis_index("s") * NUM_CORES + lax.axis_index("c")
        pltpu.sync_copy(idx_hbm.at[:, pl.ds(wid * per_worker, per_worker)], idx_v)
        lanes = lax.iota(jnp.int32, 16)
        row_vecs = [
            [lanes + (hh * EMB_DIM + e0 * 16) for e0 in range(EMB_DIM // 16)]
            for hh in range(HC)
        ]

        def chunk_step(c, carry):
            gathers = [
                pltpu.async_copy(
                    table_hbm.at[idx_v.at[c * HC + hh]],
                    buf_g.at[pl.ds(hh * per_worker, per_worker)],
                    sem,
                )
                for hh in range(HC)
            ]
            for d in gathers:
                d.wait()

            def tr_step(b, carry2):
                b_vec = jnp.full((16,), b, dtype=jnp.int32)
                for hh in range(HC):
                    for e0 in range(EMB_DIM // 16):
                        vals = buf_g[hh * per_worker + b, pl.ds(e0 * 16, 16)]
                        plsc.store_scatter(buf_t, [row_vecs[hh][e0], b_vec], vals)
                return carry2

            lax.fori_loop(0, per_worker, tr_step, 0)
            pltpu.sync_copy(
                buf_t.at[:, pl.ds(0, per_worker)],
                out_hbm.at[
                    pl.ds(c * HC * EMB_DIM, HC * EMB_DIM),
                    pl.ds(wid * per_worker, per_worker),
                ],
            )
            return carry

        lax.fori_loop(0, chunks, chunk_step, 0)

    return lookup


def kernel(x, table):
    batch, hist = x.shape
    out = _make_lookup(batch, hist)(x.T, table)
    return out.reshape(hist, EMB_DIM, batch).transpose(2, 0, 1)


# speedup vs baseline: 1.1185x; 1.0008x over previous
import functools

import jax
import jax.numpy as jnp
from jax import lax
from jax.experimental import pallas as pl
from jax.experimental.pallas import tpu as pltpu
from jax.experimental.pallas import tpu_sc as plsc

EMB_DIM = 64
NUM_CORES = 2
NUM_SUBCORES = 16
NUM_WORKERS = NUM_CORES * NUM_SUBCORES
HC = 5
TPAD = 137


def _make_lookup(batch: int, hist: int):
    per_worker = batch // NUM_WORKERS
    chunks = hist // HC
    rows_per_chunk = HC * per_worker
    mesh = plsc.VectorSubcoreMesh(core_axis_name="c", subcore_axis_name="s")

    @functools.partial(
        pl.kernel,
        mesh=mesh,
        out_type=jax.ShapeDtypeStruct((hist * EMB_DIM, batch), jnp.float32),
        scratch_types=[
            pltpu.VMEM((hist, per_worker), jnp.int32),
            pltpu.VMEM((rows_per_chunk, EMB_DIM), jnp.float32),
            pltpu.VMEM((HC * EMB_DIM, TPAD), jnp.float32),
            pltpu.SemaphoreType.DMA,
        ],
        compiler_params=pltpu.CompilerParams(
            use_tc_tiling_on_sc=False, needs_layout_passes=False
        ),
    )
    def lookup(idx_hbm, table_hbm, out_hbm, idx_v, buf_g, buf_t, sem):
        wid = lax.axis_index("s") * NUM_CORES + lax.axis_index("c")
        pltpu.sync_copy(idx_hbm.at[:, pl.ds(wid * per_worker, per_worker)], idx_v)
        lanes = lax.iota(jnp.int32, 16)
        row_vecs = [
            [lanes + (hh * EMB_DIM + e0 * 16) for e0 in range(EMB_DIM // 16)]
            for hh in range(HC)
        ]

        def chunk_step(c, carry):
            gathers = [
                pltpu.async_copy(
                    table_hbm.at[idx_v.at[c * HC + hh]],
                    buf_g.at[pl.ds(hh * per_worker, per_worker)],
                    sem,
                )
                for hh in range(HC)
            ]
            for d in gathers:
                d.wait()

            def tr_step(bb, carry2):
                for u in range(4):
                    b = bb * 4 + u
                    b_vec = jnp.full((16,), b, dtype=jnp.int32)
                    for hh in range(HC):
                        for e0 in range(EMB_DIM // 16):
                            vals = buf_g[hh * per_worker + b, pl.ds(e0 * 16, 16)]
                            plsc.store_scatter(
                                buf_t, [row_vecs[hh][e0], b_vec], vals
                            )
                return carry2

            lax.fori_loop(0, per_worker // 4, tr_step, 0)
            pltpu.sync_copy(
                buf_t.at[:, pl.ds(0, per_worker)],
                out_hbm.at[
                    pl.ds(c * HC * EMB_DIM, HC * EMB_DIM),
                    pl.ds(wid * per_worker, per_worker),
                ],
            )
            return carry

        lax.fori_loop(0, chunks, chunk_step, 0)

    return lookup


def kernel(x, table):
    batch, hist = x.shape
    out = _make_lookup(batch, hist)(x.T, table)
    return out.reshape(hist, EMB_DIM, batch).transpose(2, 0, 1)


# speedup vs baseline: 1.1959x; 1.0692x over previous
import functools

import jax
import jax.numpy as jnp
from jax import lax
from jax.experimental import pallas as pl
from jax.experimental.pallas import tpu as pltpu
from jax.experimental.pallas import tpu_sc as plsc

EMB_DIM = 64
NUM_CORES = 2
NUM_SUBCORES = 16
NUM_WORKERS = NUM_CORES * NUM_SUBCORES
CHUNK = 640


def _make_lookup(total_rows: int):
    chunks_per_worker = total_rows // (NUM_WORKERS * CHUNK)
    mesh = plsc.VectorSubcoreMesh(core_axis_name="c", subcore_axis_name="s")

    @functools.partial(
        pl.kernel,
        mesh=mesh,
        out_type=jax.ShapeDtypeStruct((total_rows, EMB_DIM), jnp.float32),
        scratch_types=[
            pltpu.VMEM((chunks_per_worker, CHUNK), jnp.int32),
            pltpu.VMEM((CHUNK, EMB_DIM), jnp.float32),
            pltpu.VMEM((CHUNK, EMB_DIM), jnp.float32),
            pltpu.SemaphoreType.DMA,
            pltpu.SemaphoreType.DMA,
            pltpu.SemaphoreType.DMA,
            pltpu.SemaphoreType.DMA,
        ],
        compiler_params=pltpu.CompilerParams(use_tc_tiling_on_sc=False),
    )
    def lookup(idx_hbm, table_hbm, out_hbm, idx_v, buf0, buf1, sg0, sg1, sw0, sw1):
        wid = lax.axis_index("s") * NUM_CORES + lax.axis_index("c")
        pltpu.sync_copy(idx_hbm.at[wid], idx_v)
        base = wid * chunks_per_worker * CHUNK

        bufs = [buf0, buf1]
        sg = [sg0, sg1]
        sw = [sw0, sw1]
        gathers = [None, None]
        writebacks = [None, None]
        for j in range(chunks_per_worker):
            b = j % 2
            if writebacks[b] is not None:
                writebacks[b].wait()
            gathers[b] = pltpu.async_copy(table_hbm.at[idx_v.at[j]], bufs[b], sg[b])
            if j >= 1:
                pb = (j - 1) % 2
                gathers[pb].wait()
                writebacks[pb] = pltpu.async_copy(
                    bufs[pb], out_hbm.at[pl.ds(base + (j - 1) * CHUNK, CHUNK)], sw[pb]
                )
        last = (chunks_per_worker - 1) % 2
        gathers[last].wait()
        writebacks[last] = pltpu.async_copy(
            bufs[last],
            out_hbm.at[pl.ds(base + (chunks_per_worker - 1) * CHUNK, CHUNK)],
            sw[last],
        )
        writebacks[1 - last].wait()
        writebacks[last].wait()

    return lookup


def kernel(x, table):
    batch, hist = x.shape
    total = batch * hist
    chunks_per_worker = total // (NUM_WORKERS * CHUNK)
    idx3d = x.reshape(NUM_WORKERS, chunks_per_worker, CHUNK)
    out = _make_lookup(total)(idx3d, table)
    return out.reshape(batch, hist, EMB_DIM)
